# Initial kernel scaffold; baseline (speedup 1.0000x reference)
import functools

import jax
import jax.numpy as jnp
from jax import lax
from jax.experimental import pallas as pl
from jax.experimental.pallas import tpu as pltpu
from jax.experimental.pallas import tpu_sc as plsc

N = 10000
E = 320000
D = 128
B = 64
OUT = 128

NW = 32
EW = E // NW
CH = 128
NCH = 79
EWP = NCH * CH
NCHN = 79
NPAD = NCHN * CH
TRASH = N
MROWS = 10240
STRIPE = MROWS // 16

_mesh = plsc.VectorSubcoreMesh(core_axis_name="c", subcore_axis_name="s")
_f32 = jnp.float32


def _zero_rows(buf):
    zv = jnp.zeros((16,), _f32)

    def body(r, carry):
        for k in range(D // 16):
            buf[r, pl.ds(16 * k, 16)] = zv
        return carry

    lax.fori_loop(0, CH, body, 0)


def _sc_prep_body(f0_hbm, f1_hbm, src_hbm, dst_hbm, emb0_hbm, emb1_hbm,
                  hv_hbm, degs_hbm,
                  dout_sp, din_sp, zb, onesb, i0v, i1v, rows, rows2,
                  sem0, sem1):
    cid = lax.axis_index("c")
    sid = lax.axis_index("s")
    wid = cid * 16 + sid

    zv = jnp.zeros((16,), _f32)
    ov = jnp.ones((16,), _f32)
    for k in range(STRIPE // 16):
        zb[pl.ds(16 * k, 16)] = zv
    for k in range(CH // 16):
        onesb[pl.ds(16 * k, 16)] = ov

    pltpu.sync_copy(zb, dout_sp.at[pl.ds(STRIPE * sid, STRIPE)])
    pltpu.sync_copy(zb, din_sp.at[pl.ds(STRIPE * sid, STRIPE)])
    plsc.subcore_barrier()

    def deg_body(j, carry):
        pltpu.sync_copy(src_hbm.at[wid, j], i0v)
        pltpu.sync_copy(dst_hbm.at[wid, j], i1v)
        pltpu.sync_copy(onesb, dout_sp.at[i0v], add=True)
        pltpu.sync_copy(onesb, din_sp.at[i1v], add=True)
        return carry

    lax.fori_loop(0, NCH, deg_body, 0)

    def emb_body(t, carry):
        j = wid + NW * t

        @pl.when(j < NCHN)
        def _():
            pltpu.sync_copy(f0_hbm.at[j], i0v)
            pltpu.sync_copy(f1_hbm.at[j], i1v)
            pltpu.async_copy(emb0_hbm.at[i0v], rows, sem0).wait()
            pltpu.async_copy(emb1_hbm.at[i1v], rows2, sem1).wait()

            def addrow(r, c2):
                for k in range(D // 16):
                    sl = pl.ds(16 * k, 16)
                    rows[r, sl] = rows[r, sl] + rows2[r, sl]
                return c2

            lax.fori_loop(0, CH, addrow, 0)
            pltpu.sync_copy(rows, hv_hbm.at[j])

        return carry

    lax.fori_loop(0, (NCHN + NW - 1) // NW, emb_body, 0)

    plsc.subcore_barrier()
    sl = pl.ds(STRIPE * sid, STRIPE)
    pltpu.sync_copy(dout_sp.at[sl], degs_hbm.at[cid, 0, sl])
    pltpu.sync_copy(din_sp.at[sl], degs_hbm.at[cid, 1, sl])


_sc_prep = functools.partial(
    pl.kernel,
    out_type=[
        jax.ShapeDtypeStruct((NCHN, CH, D), _f32),
        jax.ShapeDtypeStruct((2, 2, MROWS), _f32),
    ],
    mesh=_mesh,
    scratch_types=[
        pltpu.VMEM_SHARED((MROWS,), _f32),
        pltpu.VMEM_SHARED((MROWS,), _f32),
        pltpu.VMEM((STRIPE,), _f32),
        pltpu.VMEM((CH,), _f32),
        pltpu.VMEM((CH,), jnp.int32),
        pltpu.VMEM((CH,), jnp.int32),
        pltpu.VMEM((CH, D), _f32),
        pltpu.VMEM((CH, D), _f32),
        pltpu.SemaphoreType.DMA,
        pltpu.SemaphoreType.DMA,
    ],
)(_sc_prep_body)


def _sc_msg_body(h_hbm, src_hbm, dst_hbm, out_hbm,
                 m_sp, srcv, dstv, rows, sem):
    cid = lax.axis_index("c")
    sid = lax.axis_index("s")
    wid = cid * 16 + sid

    _zero_rows(rows)
    for t in range(STRIPE // CH):
        pltpu.sync_copy(rows, m_sp.at[pl.ds(STRIPE * sid + CH * t, CH)])

    pltpu.sync_copy(src_hbm.at[wid], srcv)
    pltpu.sync_copy(dst_hbm.at[wid], dstv)
    plsc.subcore_barrier()

    def chunk_body(j, carry):
        pltpu.async_copy(h_hbm.at[srcv.at[j]], rows, sem).wait()
        pltpu.sync_copy(rows, m_sp.at[dstv.at[j]], add=True)
        return carry

    lax.fori_loop(0, NCH, chunk_body, 0)

    plsc.subcore_barrier()
    sl = pl.ds(STRIPE * sid, STRIPE)
    pltpu.sync_copy(m_sp.at[sl], out_hbm.at[cid, sl])


_sc_msg = functools.partial(
    pl.kernel,
    out_type=jax.ShapeDtypeStruct((2, MROWS, D), _f32),
    mesh=_mesh,
    scratch_types=[
        pltpu.VMEM_SHARED((MROWS, D), _f32),
        pltpu.VMEM((NCH, CH), jnp.int32),
        pltpu.VMEM((NCH, CH), jnp.int32),
        pltpu.VMEM((CH, D), _f32),
        pltpu.SemaphoreType.DMA,
    ],
)(_sc_msg_body)


def _tc_prep_body(dego_ref, degi_ref, hv_ref, h1_ref, nin_ref, nout_ref):
    do_ = dego_ref[0] + dego_ref[1]
    di = degi_ref[0] + degi_ref[1]
    no = lax.rsqrt(jnp.maximum(do_, 1.0))
    ni = lax.rsqrt(jnp.maximum(di, 1.0))
    nout_ref[...] = no
    nin_ref[...] = ni
    h1_ref[...] = hv_ref[...] * no


def _tc_dense_body(p_ref, nin_ref, nout_ref, w_ref, b_ref, h_ref):
    m = (p_ref[0] + p_ref[1]) * nin_ref[...]
    h = jnp.dot(m, w_ref[...], preferred_element_type=_f32) + b_ref[...]
    h_ref[...] = jnp.maximum(h, 0.0) * nout_ref[...]


def _tc_final_body(p_ref, nin_ref, gid_ref, w_ref, b_ref, wout_ref, bout_ref,
                   out_ref, acc, cnt):
    i = pl.program_id(0)
    m = (p_ref[0] + p_ref[1]) * nin_ref[...]
    h = jnp.maximum(
        jnp.dot(m, w_ref[...], preferred_element_type=_f32) + b_ref[...], 0.0)
    gid = gid_ref[...][:, 0]
    oh = (lax.broadcasted_iota(jnp.int32, (B, CH), 0) == gid[None, :]).astype(_f32)
    part = jnp.dot(oh, h, preferred_element_type=_f32)
    pcnt = jnp.sum(oh, axis=1, keepdims=True)

    @pl.when(i == 0)
    def _():
        acc[...] = part
        cnt[...] = pcnt

    @pl.when(i > 0)
    def _():
        acc[...] += part
        cnt[...] += pcnt

    @pl.when(i == NCHN - 1)
    def _():
        hg = acc[...] / jnp.maximum(cnt[...], 1.0)
        out_ref[...] = (
            jnp.dot(hg, wout_ref[...], preferred_element_type=_f32)
            + bout_ref[...])


def _tc_prep(dego, degi, hv):
    return pl.pallas_call(
        _tc_prep_body,
        grid=(NCHN,),
        in_specs=[
            pl.BlockSpec((2, CH, 1), lambda i: (0, i, 0)),
            pl.BlockSpec((2, CH, 1), lambda i: (0, i, 0)),
            pl.BlockSpec((CH, D), lambda i: (i, 0)),
        ],
        out_specs=[
            pl.BlockSpec((CH, D), lambda i: (i, 0)),
            pl.BlockSpec((CH, 1), lambda i: (i, 0)),
            pl.BlockSpec((CH, 1), lambda i: (i, 0)),
        ],
        out_shape=[
            jax.ShapeDtypeStruct((NPAD, D), _f32),
            jax.ShapeDtypeStruct((NPAD, 1), _f32),
            jax.ShapeDtypeStruct((NPAD, 1), _f32),
        ],
    )(dego, degi, hv)


def _tc_dense(p, nin, nout, w, b):
    return pl.pallas_call(
        _tc_dense_body,
        grid=(NCHN,),
        in_specs=[
            pl.BlockSpec((2, CH, D), lambda i: (0, i, 0)),
            pl.BlockSpec((CH, 1), lambda i: (i, 0)),
            pl.BlockSpec((CH, 1), lambda i: (i, 0)),
            pl.BlockSpec((D, D), lambda i: (0, 0)),
            pl.BlockSpec((1, D), lambda i: (0, 0)),
        ],
        out_specs=pl.BlockSpec((CH, D), lambda i: (i, 0)),
        out_shape=jax.ShapeDtypeStruct((NPAD, D), _f32),
    )(p, nin, nout, w, b)


def _tc_final(p, nin, gid, w, b, wout, bout):
    return pl.pallas_call(
        _tc_final_body,
        grid=(NCHN,),
        in_specs=[
            pl.BlockSpec((2, CH, D), lambda i: (0, i, 0)),
            pl.BlockSpec((CH, 1), lambda i: (i, 0)),
            pl.BlockSpec((CH, 1), lambda i: (i, 0)),
            pl.BlockSpec((D, D), lambda i: (0, 0)),
            pl.BlockSpec((1, D), lambda i: (0, 0)),
            pl.BlockSpec((D, OUT), lambda i: (0, 0)),
            pl.BlockSpec((1, OUT), lambda i: (0, 0)),
        ],
        out_specs=pl.BlockSpec((B, OUT), lambda i: (0, 0)),
        out_shape=jax.ShapeDtypeStruct((B, OUT), _f32),
        scratch_shapes=[
            pltpu.VMEM((B, D), _f32),
            pltpu.VMEM((B, 1), _f32),
        ],
    )(p, nin, gid, w, b, wout, bout)


def kernel(feats0, feats1, edge_index, graph_ids,
           emb0, emb1, W0, b0, W1, b1, W2, b2, Wout, bout):
    src = edge_index[0].reshape(NW, EW)
    dst = edge_index[1].reshape(NW, EW)
    srcp = jnp.pad(src, ((0, 0), (0, EWP - EW))).reshape(NW, NCH, CH)
    dstp = jnp.pad(dst, ((0, 0), (0, EWP - EW)),
                   constant_values=TRASH).reshape(NW, NCH, CH)
    f0p = jnp.pad(feats0, (0, NPAD - N)).reshape(NCHN, CH)
    f1p = jnp.pad(feats1, (0, NPAD - N)).reshape(NCHN, CH)
    gidp = jnp.pad(graph_ids, (0, NPAD - N),
                   constant_values=B).reshape(NPAD, 1)

    hv_pad, degs = _sc_prep(f0p, f1p, srcp, dstp, emb0, emb1)
    hv = hv_pad.reshape(NPAD, D)
    dego = degs[:, 0, :NPAD].reshape(2, NPAD, 1)
    degi = degs[:, 1, :NPAD].reshape(2, NPAD, 1)

    h, nin, nout = _tc_prep(dego, degi, hv)
    for w, b_ in ((W0, b0), (W1, b1)):
        p = _sc_msg(h[:N], srcp, dstp)
        h = _tc_dense(p, nin, nout, w, b_.reshape(1, D))
    p = _sc_msg(h[:N], srcp, dstp)
    return _tc_final(p, nin, gidp, W2, b2.reshape(1, D),
                     Wout, bout.reshape(1, OUT))


# speedup vs baseline: 4.0823x; 4.0823x over previous
import functools

import jax
import jax.numpy as jnp
from jax import lax
from jax.experimental import pallas as pl
from jax.experimental.pallas import tpu as pltpu
from jax.experimental.pallas import tpu_sc as plsc

N = 10000
E = 320000
D = 128
B = 64
OUT = 128

NW = 32
EW = E // NW
CH = 128
NCH = 79
EWP = NCH * CH
NCHN = 79
NPAD = NCHN * CH
TRASH = N
MROWS = 10240
STRIPE = MROWS // 16

_mesh = plsc.VectorSubcoreMesh(core_axis_name="c", subcore_axis_name="s")
_f32 = jnp.float32


def _zero_rows(buf):
    zv = jnp.zeros((16,), _f32)

    def body(r, carry):
        for k in range(D // 16):
            buf[r, pl.ds(16 * k, 16)] = zv
        return carry

    lax.fori_loop(0, CH, body, 0)


def _sc_prep_body(f0_hbm, f1_hbm, src_hbm, dst_hbm, emb0_hbm, emb1_hbm,
                  hv_hbm, degs_hbm,
                  dout_sp, din_sp, zb, onesb, i0v, i1v, rows, rows2,
                  sem0, sem1):
    cid = lax.axis_index("c")
    sid = lax.axis_index("s")
    wid = cid * 16 + sid

    zv = jnp.zeros((16,), _f32)
    ov = jnp.ones((16,), _f32)
    for k in range(STRIPE // 16):
        zb[pl.ds(16 * k, 16)] = zv
    for k in range(CH // 16):
        onesb[pl.ds(16 * k, 16)] = ov

    pltpu.sync_copy(zb, dout_sp.at[pl.ds(STRIPE * sid, STRIPE)])
    pltpu.sync_copy(zb, din_sp.at[pl.ds(STRIPE * sid, STRIPE)])
    plsc.subcore_barrier()

    def deg_body(j, carry):
        pltpu.sync_copy(src_hbm.at[wid, j], i0v)
        pltpu.sync_copy(dst_hbm.at[wid, j], i1v)
        pltpu.sync_copy(onesb, dout_sp.at[i0v], add=True)
        pltpu.sync_copy(onesb, din_sp.at[i1v], add=True)
        return carry

    lax.fori_loop(0, NCH, deg_body, 0)

    def emb_body(t, carry):
        j = wid + NW * t

        @pl.when(j < NCHN)
        def _():
            pltpu.sync_copy(f0_hbm.at[j], i0v)
            pltpu.sync_copy(f1_hbm.at[j], i1v)
            pltpu.async_copy(emb0_hbm.at[i0v], rows, sem0).wait()
            pltpu.async_copy(emb1_hbm.at[i1v], rows2, sem1).wait()

            def addrow(r, c2):
                for k in range(D // 16):
                    sl = pl.ds(16 * k, 16)
                    rows[r, sl] = rows[r, sl] + rows2[r, sl]
                return c2

            lax.fori_loop(0, CH, addrow, 0)
            pltpu.sync_copy(rows, hv_hbm.at[j])

        return carry

    lax.fori_loop(0, (NCHN + NW - 1) // NW, emb_body, 0)

    plsc.subcore_barrier()
    sl = pl.ds(STRIPE * sid, STRIPE)
    pltpu.sync_copy(dout_sp.at[sl], degs_hbm.at[cid, 0, sl])
    pltpu.sync_copy(din_sp.at[sl], degs_hbm.at[cid, 1, sl])


_sc_prep = functools.partial(
    pl.kernel,
    out_type=[
        jax.ShapeDtypeStruct((NCHN, CH, D), _f32),
        jax.ShapeDtypeStruct((2, 2, MROWS), _f32),
    ],
    mesh=_mesh,
    scratch_types=[
        pltpu.VMEM_SHARED((MROWS,), _f32),
        pltpu.VMEM_SHARED((MROWS,), _f32),
        pltpu.VMEM((STRIPE,), _f32),
        pltpu.VMEM((CH,), _f32),
        pltpu.VMEM((CH,), jnp.int32),
        pltpu.VMEM((CH,), jnp.int32),
        pltpu.VMEM((CH, D), _f32),
        pltpu.VMEM((CH, D), _f32),
        pltpu.SemaphoreType.DMA,
        pltpu.SemaphoreType.DMA,
    ],
)(_sc_prep_body)


def _sc_msg_body(h_hbm, src_hbm, dst_hbm, out_hbm,
                 m_sp, srcv, dstv, rows, sem):
    cid = lax.axis_index("c")
    sid = lax.axis_index("s")
    wid = cid * 16 + sid

    _zero_rows(rows)
    for t in range(STRIPE // CH):
        pltpu.sync_copy(rows, m_sp.at[pl.ds(STRIPE * sid + CH * t, CH)])

    pltpu.sync_copy(src_hbm.at[wid], srcv)
    pltpu.sync_copy(dst_hbm.at[wid], dstv)
    plsc.subcore_barrier()

    def chunk_body(j, carry):
        pltpu.async_copy(h_hbm.at[srcv.at[j]], rows, sem).wait()
        pltpu.sync_copy(rows, m_sp.at[dstv.at[j]], add=True)
        return carry

    lax.fori_loop(0, NCH, chunk_body, 0)

    plsc.subcore_barrier()
    sl = pl.ds(STRIPE * sid, STRIPE)
    pltpu.sync_copy(m_sp.at[sl], out_hbm.at[cid, sl])


_sc_msg = functools.partial(
    pl.kernel,
    out_type=jax.ShapeDtypeStruct((2, MROWS, D), _f32),
    mesh=_mesh,
    scratch_types=[
        pltpu.VMEM_SHARED((MROWS, D), _f32),
        pltpu.VMEM((NCH, CH), jnp.int32),
        pltpu.VMEM((NCH, CH), jnp.int32),
        pltpu.VMEM((CH, D), _f32),
        pltpu.SemaphoreType.DMA,
    ],
)(_sc_msg_body)


def _tc_prep_body(dego_ref, degi_ref, hv_ref, h1_ref, nin_ref, nout_ref):
    do_ = dego_ref[0] + dego_ref[1]
    di = degi_ref[0] + degi_ref[1]
    no = lax.rsqrt(jnp.maximum(do_, 1.0))
    ni = lax.rsqrt(jnp.maximum(di, 1.0))
    nout_ref[...] = no
    nin_ref[...] = ni
    h1_ref[...] = hv_ref[...] * no


def _tc_dense_body(p_ref, nin_ref, nout_ref, w_ref, b_ref, h_ref):
    m = (p_ref[0] + p_ref[1]) * nin_ref[...]
    h = jnp.dot(m, w_ref[...], preferred_element_type=_f32) + b_ref[...]
    h_ref[...] = jnp.maximum(h, 0.0) * nout_ref[...]


def _tc_final_body(p_ref, nin_ref, gid_ref, w_ref, b_ref, wout_ref, bout_ref,
                   out_ref, acc, cnt):
    i = pl.program_id(0)
    m = (p_ref[0] + p_ref[1]) * nin_ref[...]
    h = jnp.maximum(
        jnp.dot(m, w_ref[...], preferred_element_type=_f32) + b_ref[...], 0.0)
    gid = gid_ref[...][:, 0]
    oh = (lax.broadcasted_iota(jnp.int32, (B, CH), 0) == gid[None, :]).astype(_f32)
    part = jnp.dot(oh, h, preferred_element_type=_f32)
    pcnt = jnp.sum(oh, axis=1, keepdims=True)

    @pl.when(i == 0)
    def _():
        acc[...] = part
        cnt[...] = pcnt

    @pl.when(i > 0)
    def _():
        acc[...] += part
        cnt[...] += pcnt

    @pl.when(i == NCHN - 1)
    def _():
        hg = acc[...] / jnp.maximum(cnt[...], 1.0)
        out_ref[...] = (
            jnp.dot(hg, wout_ref[...], preferred_element_type=_f32)
            + bout_ref[...])


def _tc_prep(dego, degi, hv):
    return pl.pallas_call(
        _tc_prep_body,
        grid=(NCHN,),
        in_specs=[
            pl.BlockSpec((2, CH, 1), lambda i: (0, i, 0)),
            pl.BlockSpec((2, CH, 1), lambda i: (0, i, 0)),
            pl.BlockSpec((CH, D), lambda i: (i, 0)),
        ],
        out_specs=[
            pl.BlockSpec((CH, D), lambda i: (i, 0)),
            pl.BlockSpec((CH, 1), lambda i: (i, 0)),
            pl.BlockSpec((CH, 1), lambda i: (i, 0)),
        ],
        out_shape=[
            jax.ShapeDtypeStruct((NPAD, D), _f32),
            jax.ShapeDtypeStruct((NPAD, 1), _f32),
            jax.ShapeDtypeStruct((NPAD, 1), _f32),
        ],
    )(dego, degi, hv)


def _tc_dense(p, nin, nout, w, b):
    return pl.pallas_call(
        _tc_dense_body,
        grid=(NCHN,),
        in_specs=[
            pl.BlockSpec((2, CH, D), lambda i: (0, i, 0)),
            pl.BlockSpec((CH, 1), lambda i: (i, 0)),
            pl.BlockSpec((CH, 1), lambda i: (i, 0)),
            pl.BlockSpec((D, D), lambda i: (0, 0)),
            pl.BlockSpec((1, D), lambda i: (0, 0)),
        ],
        out_specs=pl.BlockSpec((CH, D), lambda i: (i, 0)),
        out_shape=jax.ShapeDtypeStruct((NPAD, D), _f32),
    )(p, nin, nout, w, b)


def _tc_final(p, nin, gid, w, b, wout, bout):
    return pl.pallas_call(
        _tc_final_body,
        grid=(NCHN,),
        in_specs=[
            pl.BlockSpec((2, CH, D), lambda i: (0, i, 0)),
            pl.BlockSpec((CH, 1), lambda i: (i, 0)),
            pl.BlockSpec((CH, 1), lambda i: (i, 0)),
            pl.BlockSpec((D, D), lambda i: (0, 0)),
            pl.BlockSpec((1, D), lambda i: (0, 0)),
            pl.BlockSpec((D, OUT), lambda i: (0, 0)),
            pl.BlockSpec((1, OUT), lambda i: (0, 0)),
        ],
        out_specs=pl.BlockSpec((B, OUT), lambda i: (0, 0)),
        out_shape=jax.ShapeDtypeStruct((B, OUT), _f32),
        scratch_shapes=[
            pltpu.VMEM((B, D), _f32),
            pltpu.VMEM((B, 1), _f32),
        ],
    )(p, nin, gid, w, b, wout, bout)


def kernel(feats0, feats1, edge_index, graph_ids,
           emb0, emb1, W0, b0, W1, b1, W2, b2, Wout, bout):
    src = edge_index[0].reshape(NW, EW)
    dst = edge_index[1].reshape(NW, EW)
    srcp = jnp.pad(src, ((0, 0), (0, EWP - EW))).reshape(NW, NCH, CH)
    dstp = jnp.pad(dst, ((0, 0), (0, EWP - EW)),
                   constant_values=TRASH).reshape(NW, NCH, CH)
    f0p = jnp.pad(feats0, (0, NPAD - N)).reshape(NCHN, CH)
    f1p = jnp.pad(feats1, (0, NPAD - N)).reshape(NCHN, CH)
    gidp = jnp.pad(graph_ids, (0, NPAD - N),
                   constant_values=B).reshape(NPAD, 1)

    hv_pad, degs = _sc_prep(f0p, f1p, srcp, dstp, emb0, emb1)
    hv = hv_pad.reshape(NPAD, D)
    dego = degs[:, 0, :NPAD].reshape(2, NPAD, 1)
    degi = degs[:, 1, :NPAD].reshape(2, NPAD, 1)

    h, nin, nout = _tc_prep(dego, degi, hv)
    for w, b_ in ((W0, b0), (W1, b1)):
        p = _sc_msg(h, srcp, dstp)
        h = _tc_dense(p, nin, nout, w, b_.reshape(1, D))
    p = _sc_msg(h, srcp, dstp)
    return _tc_final(p, nin, gidp, W2, b2.reshape(1, D),
                     Wout, bout.reshape(1, OUT))
